# baseline (speedup 1.0000x reference)
import jax
import jax.numpy as jnp
from jax.experimental import pallas as pl
from jax.experimental.pallas import tpu as pltpu

H = 768
NH = 12
DH = H // NH
E = 8
I = 1536

SQ = 2048
SI = 1024

_BT = 256
_BTM = 512
_NTQ = SQ // _BTM
_NT = (SQ + SI) // _BTM

_BF = jnp.bfloat16
_CT = (((1,), (1,)), ((), ()))


def _lnmm_kernel(x_ref, s_ref, b_ref, w_ref, bias_ref, o_ref, wb_ref):
    @pl.when(pl.program_id(0) == 0)
    def _():
        wb_ref[...] = w_ref[...].astype(_BF)

    x = x_ref[...]
    mu = jnp.mean(x, axis=1, keepdims=True)
    var = jnp.mean((x - mu) ** 2, axis=1, keepdims=True)
    xn = (x - mu) / jnp.sqrt(var + 1e-5) * s_ref[...] + b_ref[...]
    acc = jax.lax.dot_general(
        xn.astype(_BF), wb_ref[...], _CT,
        preferred_element_type=jnp.float32) + bias_ref[...]
    o_ref[...] = acc.astype(_BF)


def _mm_kernel(x_ref, w_ref, bias_ref, o_ref, wb_ref):
    @pl.when(pl.program_id(0) == 0)
    def _():
        wb_ref[...] = w_ref[...].astype(_BF)

    acc = jax.lax.dot_general(
        x_ref[...].astype(_BF), wb_ref[...], _CT,
        preferred_element_type=jnp.float32) + bias_ref[...]
    o_ref[...] = acc.astype(_BF)


def _mmres_kernel(x_ref, w_ref, bias_ref, res_ref, o_ref, wb_ref):
    @pl.when(pl.program_id(0) == 0)
    def _():
        wb_ref[...] = w_ref[...].astype(_BF)

    o_ref[...] = (
        jax.lax.dot_general(x_ref[...], wb_ref[...], _CT,
                            preferred_element_type=jnp.float32)
        + bias_ref[...]
        + res_ref[...]
    )


def _mmres_lnmm_kernel(x_ref, w_ref, bias_ref, res_ref, s_ref, b_ref,
                       w2_ref, bias2_ref, o1_ref, o2_ref, wb_ref, w2b_ref):
    @pl.when(pl.program_id(0) == 0)
    def _():
        wb_ref[...] = w_ref[...].astype(_BF)
        w2b_ref[...] = w2_ref[...].astype(_BF)

    r = (
        jax.lax.dot_general(x_ref[...], wb_ref[...], _CT,
                            preferred_element_type=jnp.float32)
        + bias_ref[...]
        + res_ref[...]
    )
    o1_ref[...] = r
    mu = jnp.mean(r, axis=1, keepdims=True)
    var = jnp.mean((r - mu) ** 2, axis=1, keepdims=True)
    xn = (r - mu) / jnp.sqrt(var + 1e-5) * s_ref[...] + b_ref[...]
    acc = jax.lax.dot_general(
        xn.astype(_BF), w2b_ref[...], _CT,
        preferred_element_type=jnp.float32) + bias2_ref[...]
    o2_ref[...] = acc.astype(_BF)


def _attn_kernel(q_ref, k_ref, v_ref, o_ref):
    q = q_ref[...]
    k = k_ref[...]
    v = v_ref[...]
    scale = 1.0 / float(DH) ** 0.5
    for h in range(NH):
        lo, hi = h * DH, (h + 1) * DH
        qh = q[:, lo:hi]
        kh = k[:, lo:hi]
        vh = v[:, lo:hi]
        s = jax.lax.dot_general(
            qh, kh, _CT, preferred_element_type=jnp.float32) * scale
        p = jnp.exp(s - 12.0)
        inv = 1.0 / jnp.sum(p, axis=1, keepdims=True)
        o = jnp.dot(p.astype(_BF), vh, preferred_element_type=jnp.float32)
        o_ref[:, lo:hi] = (o * inv).astype(_BF)


def _gate_kernel(img_ref, txt_ref, q2_ref, iga_ref, igb_ref, igbias_ref,
                 tga_ref, tgb_ref, tgbias_ref, lns_ref, lnb_ref,
                 tp_ref, ip_ref, qffn_ref):
    img = img_ref[...]
    q2 = q2_ref[...]
    ictx = jnp.mean(img, axis=0, keepdims=True)
    tctx = jnp.mean(txt_ref[...], axis=0, keepdims=True)
    il = (
        jnp.dot(img.astype(_BF), iga_ref[...],
                preferred_element_type=jnp.float32)
        + jnp.dot(tctx.astype(_BF), igb_ref[...],
                  preferred_element_type=jnp.float32)
        + igbias_ref[...]
    )
    tl = (
        jnp.dot(q2.astype(_BF), tga_ref[...],
                preferred_element_type=jnp.float32)
        + jnp.dot(ictx.astype(_BF), tgb_ref[...],
                  preferred_element_type=jnp.float32)
        + tgbias_ref[...]
    )

    def _softmax(x):
        m = jnp.max(x, axis=1, keepdims=True)
        e = jnp.exp(x - m)
        return e / jnp.sum(e, axis=1, keepdims=True)

    ip_ref[...] = _softmax(il)
    tp_ref[...] = _softmax(tl)
    mu = jnp.mean(q2, axis=1, keepdims=True)
    var = jnp.mean((q2 - mu) ** 2, axis=1, keepdims=True)
    qffn = (q2 - mu) / jnp.sqrt(var + 1e-5) * lns_ref[...] + lnb_ref[...]
    qffn_ref[...] = qffn.astype(_BF)


def _moe_kernel(xq_ref, xi_ref, rq_ref, ri_ref, pq_ref, pi_ref,
                w1_ref, b1_ref, w2_ref, b2_ref,
                oq_ref, oi_ref, w1b_ref, w2b_ref):
    e = pl.program_id(0)
    w1b_ref[...] = w1_ref[0].astype(_BF)
    w2b_ref[...] = w2_ref[0].astype(_BF)
    iota = jax.lax.broadcasted_iota(jnp.int32, (E, 1), 0)
    onehot = (iota == e).astype(jnp.float32)

    for t in range(_NT):
        if t < _NTQ:
            lo, hi = t * _BTM, (t + 1) * _BTM
            x = xq_ref[lo:hi, :]
            probs = pq_ref[lo:hi, :]
            r_ref, o_ref = rq_ref, oq_ref
        else:
            lo, hi = (t - _NTQ) * _BTM, (t - _NTQ + 1) * _BTM
            x = xi_ref[lo:hi, :]
            probs = pi_ref[lo:hi, :]
            r_ref, o_ref = ri_ref, oi_ref
        h = jax.lax.dot_general(
            x, w1b_ref[...], _CT,
            preferred_element_type=jnp.float32) + b1_ref[0]
        h = jax.nn.gelu(h.astype(_BF))
        y = jax.lax.dot_general(
            h, w2b_ref[...], _CT,
            preferred_element_type=jnp.float32) + b2_ref[0]
        p = jnp.dot(probs, onehot, preferred_element_type=jnp.float32)
        contrib = p * y

        @pl.when(e == 0)
        def _():
            o_ref[lo:hi, :] = r_ref[lo:hi, :] + contrib

        @pl.when(e != 0)
        def _():
            o_ref[lo:hi, :] += contrib


def _lnmm(x, s, b, w, bias):
    n, k = x.shape
    m = w.shape[0]
    return pl.pallas_call(
        _lnmm_kernel,
        grid=(n // _BT,),
        in_specs=[
            pl.BlockSpec((_BT, k), lambda i: (i, 0)),
            pl.BlockSpec((1, k), lambda i: (0, 0)),
            pl.BlockSpec((1, k), lambda i: (0, 0)),
            pl.BlockSpec((m, k), lambda i: (0, 0)),
            pl.BlockSpec((1, m), lambda i: (0, 0)),
        ],
        out_specs=pl.BlockSpec((_BT, m), lambda i: (i, 0)),
        out_shape=jax.ShapeDtypeStruct((n, m), _BF),
        scratch_shapes=[pltpu.VMEM((m, k), _BF)],
    )(x, s[None], b[None], w, bias[None])


def _mm(x, w, bias):
    n, k = x.shape
    m = w.shape[0]
    return pl.pallas_call(
        _mm_kernel,
        grid=(n // _BT,),
        in_specs=[
            pl.BlockSpec((_BT, k), lambda i: (i, 0)),
            pl.BlockSpec((m, k), lambda i: (0, 0)),
            pl.BlockSpec((1, m), lambda i: (0, 0)),
        ],
        out_specs=pl.BlockSpec((_BT, m), lambda i: (i, 0)),
        out_shape=jax.ShapeDtypeStruct((n, m), _BF),
        scratch_shapes=[pltpu.VMEM((m, k), _BF)],
    )(x, w, bias[None])


def _mmres(x, w, bias, res):
    n, k = x.shape
    return pl.pallas_call(
        _mmres_kernel,
        grid=(n // _BT,),
        in_specs=[
            pl.BlockSpec((_BT, k), lambda i: (i, 0)),
            pl.BlockSpec((H, k), lambda i: (0, 0)),
            pl.BlockSpec((1, H), lambda i: (0, 0)),
            pl.BlockSpec((_BT, H), lambda i: (i, 0)),
        ],
        out_specs=pl.BlockSpec((_BT, H), lambda i: (i, 0)),
        out_shape=jax.ShapeDtypeStruct((n, H), jnp.float32),
        scratch_shapes=[pltpu.VMEM((H, k), _BF)],
    )(x, w, bias[None], res)


def _mmres_lnmm(x, w, bias, res, s, b, w2, bias2):
    n, k = x.shape
    return pl.pallas_call(
        _mmres_lnmm_kernel,
        grid=(n // _BT,),
        in_specs=[
            pl.BlockSpec((_BT, k), lambda i: (i, 0)),
            pl.BlockSpec((H, k), lambda i: (0, 0)),
            pl.BlockSpec((1, H), lambda i: (0, 0)),
            pl.BlockSpec((_BT, H), lambda i: (i, 0)),
            pl.BlockSpec((1, H), lambda i: (0, 0)),
            pl.BlockSpec((1, H), lambda i: (0, 0)),
            pl.BlockSpec((H, H), lambda i: (0, 0)),
            pl.BlockSpec((1, H), lambda i: (0, 0)),
        ],
        out_specs=[
            pl.BlockSpec((_BT, H), lambda i: (i, 0)),
            pl.BlockSpec((_BT, H), lambda i: (i, 0)),
        ],
        out_shape=[
            jax.ShapeDtypeStruct((n, H), jnp.float32),
            jax.ShapeDtypeStruct((n, H), _BF),
        ],
        scratch_shapes=[pltpu.VMEM((H, H), _BF), pltpu.VMEM((H, H), _BF)],
    )(x, w, bias[None], res, s[None], b[None], w2, bias2[None])


def _attn(q_arr, q_idx, kv_arr, k_idx, v_idx, sk):
    n = q_arr.shape[0]
    return pl.pallas_call(
        _attn_kernel,
        grid=(n // _BT,),
        in_specs=[
            pl.BlockSpec((_BT, H), lambda i: (i, q_idx)),
            pl.BlockSpec((sk, H), lambda i: (0, k_idx)),
            pl.BlockSpec((sk, H), lambda i: (0, v_idx)),
        ],
        out_specs=pl.BlockSpec((_BT, H), lambda i: (i, 0)),
        out_shape=jax.ShapeDtypeStruct((n, H), _BF),
    )(q_arr, kv_arr, kv_arr)


def _gate(img, txt, q2, ig_w, ig_b, tg_w, tg_b, ln_s, ln_b):
    si = img.shape[0]
    sq = q2.shape[0]
    return pl.pallas_call(
        _gate_kernel,
        out_shape=[
            jax.ShapeDtypeStruct((sq, E), jnp.float32),
            jax.ShapeDtypeStruct((si, E), jnp.float32),
            jax.ShapeDtypeStruct((sq, H), _BF),
        ],
    )(img, txt, q2,
      ig_w[:, :H].T.astype(_BF), ig_w[:, H:].T.astype(_BF), ig_b[None],
      tg_w[:, :H].T.astype(_BF), tg_w[:, H:].T.astype(_BF), tg_b[None],
      ln_s[None], ln_b[None])


def _moe(xq, xi, rq, ri, pq, pi, ew1, eb1, ew2, eb2):
    return pl.pallas_call(
        _moe_kernel,
        grid=(E,),
        in_specs=[
            pl.BlockSpec((SQ, H), lambda e: (0, 0)),
            pl.BlockSpec((SI, H), lambda e: (0, 0)),
            pl.BlockSpec((SQ, H), lambda e: (0, 0)),
            pl.BlockSpec((SI, H), lambda e: (0, 0)),
            pl.BlockSpec((SQ, E), lambda e: (0, 0)),
            pl.BlockSpec((SI, E), lambda e: (0, 0)),
            pl.BlockSpec((1, I, H), lambda e: (e, 0, 0)),
            pl.BlockSpec((1, 1, I), lambda e: (e, 0, 0)),
            pl.BlockSpec((1, H, I), lambda e: (e, 0, 0)),
            pl.BlockSpec((1, 1, H), lambda e: (e, 0, 0)),
        ],
        out_specs=[
            pl.BlockSpec((SQ, H), lambda e: (0, 0)),
            pl.BlockSpec((SI, H), lambda e: (0, 0)),
        ],
        out_shape=[
            jax.ShapeDtypeStruct((SQ, H), jnp.float32),
            jax.ShapeDtypeStruct((SI, H), jnp.float32),
        ],
        scratch_shapes=[
            pltpu.VMEM((I, H), _BF),
            pltpu.VMEM((H, I), _BF),
        ],
    )(xq, xi, rq, ri, pq, pi, ew1, eb1[:, None], ew2, eb2[:, None])


def kernel(query_tokens, image_tokens, text_context, sa_w_in, sa_b_in,
           sa_w_out, sa_b_out, ca_w_in, ca_b_in, ca_w_out, ca_b_out,
           ln_q_s, ln_q_b, ln_c_s, ln_c_b, ln_f_s, ln_f_b,
           ig_w, ig_b, tg_w, tg_b, ew1, eb1, ew2, eb2):
    q = query_tokens[0]
    img = image_tokens[0]
    txt = text_context[0]

    qkv = _lnmm(q, ln_q_s, ln_q_b, sa_w_in, sa_b_in)
    attn = _attn(qkv, 0, qkv, 1, 2, q.shape[0])

    q1, cq = _mmres_lnmm(attn, sa_w_out, sa_b_out, q,
                         ln_c_s, ln_c_b, ca_w_in[:H], ca_b_in[:H])
    kv = _mm(img, ca_w_in[H:], ca_b_in[H:])
    cattn = _attn(cq, 0, kv, 0, 1, img.shape[0])
    q2 = _mmres(cattn, ca_w_out, ca_b_out, q1)

    tp, ip, qffn = _gate(img, txt, q2, ig_w, ig_b, tg_w, tg_b, ln_f_s, ln_f_b)

    q_out, img_out = _moe(qffn, img.astype(_BF), q2, img, tp, ip,
                          ew1, eb1, ew2, eb2)
    return (q_out[None], img_out[None])


# speedup vs baseline: 1.1260x; 1.1260x over previous
import jax
import jax.numpy as jnp
from jax.experimental import pallas as pl
from jax.experimental.pallas import tpu as pltpu

H = 768
NH = 12
DH = H // NH
E = 8
I = 1536

SQ = 2048
SI = 1024

_BT = 512
_BTM = 1024
_NTQ = SQ // _BTM
_NT = (SQ + SI) // _BTM

_BF = jnp.bfloat16
_CT = (((1,), (1,)), ((), ()))


def _lnmm_kernel(x_ref, s_ref, b_ref, w_ref, bias_ref, o_ref, wb_ref):
    @pl.when(pl.program_id(0) == 0)
    def _():
        wb_ref[...] = w_ref[...].astype(_BF)

    x = x_ref[...]
    mu = jnp.mean(x, axis=1, keepdims=True)
    var = jnp.mean((x - mu) ** 2, axis=1, keepdims=True)
    xn = (x - mu) / jnp.sqrt(var + 1e-5) * s_ref[...] + b_ref[...]
    acc = jax.lax.dot_general(
        xn.astype(_BF), wb_ref[...], _CT,
        preferred_element_type=jnp.float32) + bias_ref[...]
    o_ref[...] = acc.astype(_BF)


def _mm_kernel(x_ref, w_ref, bias_ref, o_ref, wb_ref):
    @pl.when(pl.program_id(0) == 0)
    def _():
        wb_ref[...] = w_ref[...].astype(_BF)

    acc = jax.lax.dot_general(
        x_ref[...].astype(_BF), wb_ref[...], _CT,
        preferred_element_type=jnp.float32) + bias_ref[...]
    o_ref[...] = acc.astype(_BF)


def _mmres_kernel(x_ref, w_ref, bias_ref, res_ref, o_ref, wb_ref):
    @pl.when(pl.program_id(0) == 0)
    def _():
        wb_ref[...] = w_ref[...].astype(_BF)

    o_ref[...] = (
        jax.lax.dot_general(x_ref[...], wb_ref[...], _CT,
                            preferred_element_type=jnp.float32)
        + bias_ref[...]
        + res_ref[...]
    )


def _mmres_lnmm_kernel(x_ref, w_ref, bias_ref, res_ref, s_ref, b_ref,
                       w2_ref, bias2_ref, o1_ref, o2_ref, wb_ref, w2b_ref):
    @pl.when(pl.program_id(0) == 0)
    def _():
        wb_ref[...] = w_ref[...].astype(_BF)
        w2b_ref[...] = w2_ref[...].astype(_BF)

    r = (
        jax.lax.dot_general(x_ref[...], wb_ref[...], _CT,
                            preferred_element_type=jnp.float32)
        + bias_ref[...]
        + res_ref[...]
    )
    o1_ref[...] = r
    mu = jnp.mean(r, axis=1, keepdims=True)
    var = jnp.mean((r - mu) ** 2, axis=1, keepdims=True)
    xn = (r - mu) / jnp.sqrt(var + 1e-5) * s_ref[...] + b_ref[...]
    acc = jax.lax.dot_general(
        xn.astype(_BF), w2b_ref[...], _CT,
        preferred_element_type=jnp.float32) + bias2_ref[...]
    o2_ref[...] = acc.astype(_BF)


def _attn_kernel(q_ref, k_ref, v_ref, o_ref):
    q = q_ref[...]
    k = k_ref[...]
    v = v_ref[...]
    scale = 1.0 / float(DH) ** 0.5
    for h in range(NH):
        lo, hi = h * DH, (h + 1) * DH
        qh = q[:, lo:hi]
        kh = k[:, lo:hi]
        vh = v[:, lo:hi]
        s = jax.lax.dot_general(
            qh, kh, _CT, preferred_element_type=jnp.float32) * scale
        p = jnp.exp(s - 12.0)
        inv = 1.0 / jnp.sum(p, axis=1, keepdims=True)
        o = jnp.dot(p.astype(_BF), vh, preferred_element_type=jnp.float32)
        o_ref[:, lo:hi] = (o * inv).astype(_BF)


def _gate_kernel(img_ref, txt_ref, q2_ref, iga_ref, igb_ref, igbias_ref,
                 tga_ref, tgb_ref, tgbias_ref, lns_ref, lnb_ref,
                 tp_ref, ip_ref, qffn_ref):
    img = img_ref[...]
    q2 = q2_ref[...]
    ictx = jnp.mean(img, axis=0, keepdims=True)
    tctx = jnp.mean(txt_ref[...], axis=0, keepdims=True)
    il = (
        jnp.dot(img.astype(_BF), iga_ref[...],
                preferred_element_type=jnp.float32)
        + jnp.dot(tctx.astype(_BF), igb_ref[...],
                  preferred_element_type=jnp.float32)
        + igbias_ref[...]
    )
    tl = (
        jnp.dot(q2.astype(_BF), tga_ref[...],
                preferred_element_type=jnp.float32)
        + jnp.dot(ictx.astype(_BF), tgb_ref[...],
                  preferred_element_type=jnp.float32)
        + tgbias_ref[...]
    )

    def _softmax(x):
        m = jnp.max(x, axis=1, keepdims=True)
        e = jnp.exp(x - m)
        return e / jnp.sum(e, axis=1, keepdims=True)

    ip_ref[...] = _softmax(il)
    tp_ref[...] = _softmax(tl)
    mu = jnp.mean(q2, axis=1, keepdims=True)
    var = jnp.mean((q2 - mu) ** 2, axis=1, keepdims=True)
    qffn = (q2 - mu) / jnp.sqrt(var + 1e-5) * lns_ref[...] + lnb_ref[...]
    qffn_ref[...] = qffn.astype(_BF)


def _moe_kernel(xq_ref, xi_ref, rq_ref, ri_ref, pq_ref, pi_ref,
                w1_ref, b1_ref, w2_ref, b2_ref,
                oq_ref, oi_ref, w1b_ref, w2b_ref):
    e = pl.program_id(0)
    w1b_ref[...] = w1_ref[0].astype(_BF)
    w2b_ref[...] = w2_ref[0].astype(_BF)
    iota = jax.lax.broadcasted_iota(jnp.int32, (E, 1), 0)
    onehot = (iota == e).astype(jnp.float32)

    for t in range(_NT):
        if t < _NTQ:
            lo, hi = t * _BTM, (t + 1) * _BTM
            x = xq_ref[lo:hi, :]
            probs = pq_ref[lo:hi, :]
            r_ref, o_ref = rq_ref, oq_ref
        else:
            lo, hi = (t - _NTQ) * _BTM, (t - _NTQ + 1) * _BTM
            x = xi_ref[lo:hi, :]
            probs = pi_ref[lo:hi, :]
            r_ref, o_ref = ri_ref, oi_ref
        h = jax.lax.dot_general(
            x, w1b_ref[...], _CT,
            preferred_element_type=jnp.float32) + b1_ref[0]
        h = jax.nn.gelu(h.astype(_BF))
        y = jax.lax.dot_general(
            h, w2b_ref[...], _CT,
            preferred_element_type=jnp.float32) + b2_ref[0]
        p = jnp.dot(probs, onehot, preferred_element_type=jnp.float32)
        contrib = p * y

        @pl.when(e == 0)
        def _():
            o_ref[lo:hi, :] = r_ref[lo:hi, :] + contrib

        @pl.when(e != 0)
        def _():
            o_ref[lo:hi, :] += contrib


def _lnmm(x, s, b, w, bias):
    n, k = x.shape
    m = w.shape[0]
    return pl.pallas_call(
        _lnmm_kernel,
        grid=(n // _BT,),
        in_specs=[
            pl.BlockSpec((_BT, k), lambda i: (i, 0)),
            pl.BlockSpec((1, k), lambda i: (0, 0)),
            pl.BlockSpec((1, k), lambda i: (0, 0)),
            pl.BlockSpec((m, k), lambda i: (0, 0)),
            pl.BlockSpec((1, m), lambda i: (0, 0)),
        ],
        out_specs=pl.BlockSpec((_BT, m), lambda i: (i, 0)),
        out_shape=jax.ShapeDtypeStruct((n, m), _BF),
        scratch_shapes=[pltpu.VMEM((m, k), _BF)],
    )(x, s[None], b[None], w, bias[None])


def _mm(x, w, bias):
    n, k = x.shape
    m = w.shape[0]
    return pl.pallas_call(
        _mm_kernel,
        grid=(n // _BT,),
        in_specs=[
            pl.BlockSpec((_BT, k), lambda i: (i, 0)),
            pl.BlockSpec((m, k), lambda i: (0, 0)),
            pl.BlockSpec((1, m), lambda i: (0, 0)),
        ],
        out_specs=pl.BlockSpec((_BT, m), lambda i: (i, 0)),
        out_shape=jax.ShapeDtypeStruct((n, m), _BF),
        scratch_shapes=[pltpu.VMEM((m, k), _BF)],
    )(x, w, bias[None])


def _mmres(x, w, bias, res):
    n, k = x.shape
    return pl.pallas_call(
        _mmres_kernel,
        grid=(n // _BT,),
        in_specs=[
            pl.BlockSpec((_BT, k), lambda i: (i, 0)),
            pl.BlockSpec((H, k), lambda i: (0, 0)),
            pl.BlockSpec((1, H), lambda i: (0, 0)),
            pl.BlockSpec((_BT, H), lambda i: (i, 0)),
        ],
        out_specs=pl.BlockSpec((_BT, H), lambda i: (i, 0)),
        out_shape=jax.ShapeDtypeStruct((n, H), jnp.float32),
        scratch_shapes=[pltpu.VMEM((H, k), _BF)],
    )(x, w, bias[None], res)


def _mmres_lnmm(x, w, bias, res, s, b, w2, bias2):
    n, k = x.shape
    return pl.pallas_call(
        _mmres_lnmm_kernel,
        grid=(n // _BT,),
        in_specs=[
            pl.BlockSpec((_BT, k), lambda i: (i, 0)),
            pl.BlockSpec((H, k), lambda i: (0, 0)),
            pl.BlockSpec((1, H), lambda i: (0, 0)),
            pl.BlockSpec((_BT, H), lambda i: (i, 0)),
            pl.BlockSpec((1, H), lambda i: (0, 0)),
            pl.BlockSpec((1, H), lambda i: (0, 0)),
            pl.BlockSpec((H, H), lambda i: (0, 0)),
            pl.BlockSpec((1, H), lambda i: (0, 0)),
        ],
        out_specs=[
            pl.BlockSpec((_BT, H), lambda i: (i, 0)),
            pl.BlockSpec((_BT, H), lambda i: (i, 0)),
        ],
        out_shape=[
            jax.ShapeDtypeStruct((n, H), jnp.float32),
            jax.ShapeDtypeStruct((n, H), _BF),
        ],
        scratch_shapes=[pltpu.VMEM((H, H), _BF), pltpu.VMEM((H, H), _BF)],
    )(x, w, bias[None], res, s[None], b[None], w2, bias2[None])


def _attn(q_arr, q_idx, kv_arr, k_idx, v_idx, sk):
    n = q_arr.shape[0]
    return pl.pallas_call(
        _attn_kernel,
        grid=(n // _BT,),
        in_specs=[
            pl.BlockSpec((_BT, H), lambda i: (i, q_idx)),
            pl.BlockSpec((sk, H), lambda i: (0, k_idx)),
            pl.BlockSpec((sk, H), lambda i: (0, v_idx)),
        ],
        out_specs=pl.BlockSpec((_BT, H), lambda i: (i, 0)),
        out_shape=jax.ShapeDtypeStruct((n, H), _BF),
    )(q_arr, kv_arr, kv_arr)


def _gate(img, txt, q2, ig_w, ig_b, tg_w, tg_b, ln_s, ln_b):
    si = img.shape[0]
    sq = q2.shape[0]
    return pl.pallas_call(
        _gate_kernel,
        out_shape=[
            jax.ShapeDtypeStruct((sq, E), jnp.float32),
            jax.ShapeDtypeStruct((si, E), jnp.float32),
            jax.ShapeDtypeStruct((sq, H), _BF),
        ],
    )(img, txt, q2,
      ig_w[:, :H].T.astype(_BF), ig_w[:, H:].T.astype(_BF), ig_b[None],
      tg_w[:, :H].T.astype(_BF), tg_w[:, H:].T.astype(_BF), tg_b[None],
      ln_s[None], ln_b[None])


def _moe(xq, xi, rq, ri, pq, pi, ew1, eb1, ew2, eb2):
    return pl.pallas_call(
        _moe_kernel,
        grid=(E,),
        in_specs=[
            pl.BlockSpec((SQ, H), lambda e: (0, 0)),
            pl.BlockSpec((SI, H), lambda e: (0, 0)),
            pl.BlockSpec((SQ, H), lambda e: (0, 0)),
            pl.BlockSpec((SI, H), lambda e: (0, 0)),
            pl.BlockSpec((SQ, E), lambda e: (0, 0)),
            pl.BlockSpec((SI, E), lambda e: (0, 0)),
            pl.BlockSpec((1, I, H), lambda e: (e, 0, 0)),
            pl.BlockSpec((1, 1, I), lambda e: (e, 0, 0)),
            pl.BlockSpec((1, H, I), lambda e: (e, 0, 0)),
            pl.BlockSpec((1, 1, H), lambda e: (e, 0, 0)),
        ],
        out_specs=[
            pl.BlockSpec((SQ, H), lambda e: (0, 0)),
            pl.BlockSpec((SI, H), lambda e: (0, 0)),
        ],
        out_shape=[
            jax.ShapeDtypeStruct((SQ, H), jnp.float32),
            jax.ShapeDtypeStruct((SI, H), jnp.float32),
        ],
        scratch_shapes=[
            pltpu.VMEM((I, H), _BF),
            pltpu.VMEM((H, I), _BF),
        ],
    )(xq, xi, rq, ri, pq, pi, ew1, eb1[:, None], ew2, eb2[:, None])


def kernel(query_tokens, image_tokens, text_context, sa_w_in, sa_b_in,
           sa_w_out, sa_b_out, ca_w_in, ca_b_in, ca_w_out, ca_b_out,
           ln_q_s, ln_q_b, ln_c_s, ln_c_b, ln_f_s, ln_f_b,
           ig_w, ig_b, tg_w, tg_b, ew1, eb1, ew2, eb2):
    q = query_tokens[0]
    img = image_tokens[0]
    txt = text_context[0]

    qkv = _lnmm(q, ln_q_s, ln_q_b, sa_w_in, sa_b_in)
    attn = _attn(qkv, 0, qkv, 1, 2, q.shape[0])

    q1, cq = _mmres_lnmm(attn, sa_w_out, sa_b_out, q,
                         ln_c_s, ln_c_b, ca_w_in[:H], ca_b_in[:H])
    kv = _mm(img, ca_w_in[H:], ca_b_in[H:])
    cattn = _attn(cq, 0, kv, 0, 1, img.shape[0])
    q2 = _mmres(cattn, ca_w_out, ca_b_out, q1)

    tp, ip, qffn = _gate(img, txt, q2, ig_w, ig_b, tg_w, tg_b, ln_f_s, ln_f_b)

    q_out, img_out = _moe(qffn, img.astype(_BF), q2, img, tp, ip,
                          ew1, eb1, ew2, eb2)
    return (q_out[None], img_out[None])


# speedup vs baseline: 1.1270x; 1.0009x over previous
import jax
import jax.numpy as jnp
from jax.experimental import pallas as pl
from jax.experimental.pallas import tpu as pltpu

H = 768
NH = 12
DH = H // NH
E = 8
I = 1536

SQ = 2048
SI = 1024

_BT = 512
_BTM = 1024
_NTQ = SQ // _BTM
_NT = (SQ + SI) // _BTM

_BF = jnp.bfloat16
_CT = (((1,), (1,)), ((), ()))


def _lnmm_kernel(x_ref, s_ref, b_ref, w_ref, bias_ref, o_ref, wb_ref):
    @pl.when(pl.program_id(0) == 0)
    def _():
        wb_ref[...] = w_ref[...].astype(_BF)

    x = x_ref[...]
    mu = jnp.mean(x, axis=1, keepdims=True)
    var = jnp.mean((x - mu) ** 2, axis=1, keepdims=True)
    xn = (x - mu) / jnp.sqrt(var + 1e-5) * s_ref[...] + b_ref[...]
    acc = jax.lax.dot_general(
        xn.astype(_BF), wb_ref[...], _CT,
        preferred_element_type=jnp.float32) + bias_ref[...]
    o_ref[...] = acc.astype(_BF)


def _mm_kernel(x_ref, w_ref, bias_ref, o_ref, wb_ref):
    @pl.when(pl.program_id(0) == 0)
    def _():
        wb_ref[...] = w_ref[...].astype(_BF)

    acc = jax.lax.dot_general(
        x_ref[...].astype(_BF), wb_ref[...], _CT,
        preferred_element_type=jnp.float32) + bias_ref[...]
    o_ref[...] = acc.astype(_BF)


def _mmres_kernel(x_ref, w_ref, bias_ref, res_ref, o_ref, wb_ref):
    @pl.when(pl.program_id(0) == 0)
    def _():
        wb_ref[...] = w_ref[...].astype(_BF)

    o_ref[...] = (
        jax.lax.dot_general(x_ref[...], wb_ref[...], _CT,
                            preferred_element_type=jnp.float32)
        + bias_ref[...]
        + res_ref[...]
    )


def _mmres_lnmm_kernel(x_ref, w_ref, bias_ref, res_ref, s_ref, b_ref,
                       w2_ref, bias2_ref, o1_ref, o2_ref, wb_ref, w2b_ref):
    @pl.when(pl.program_id(0) == 0)
    def _():
        wb_ref[...] = w_ref[...].astype(_BF)
        w2b_ref[...] = w2_ref[...].astype(_BF)

    r = (
        jax.lax.dot_general(x_ref[...], wb_ref[...], _CT,
                            preferred_element_type=jnp.float32)
        + bias_ref[...]
        + res_ref[...]
    )
    o1_ref[...] = r
    mu = jnp.mean(r, axis=1, keepdims=True)
    var = jnp.mean((r - mu) ** 2, axis=1, keepdims=True)
    xn = (r - mu) / jnp.sqrt(var + 1e-5) * s_ref[...] + b_ref[...]
    acc = jax.lax.dot_general(
        xn.astype(_BF), w2b_ref[...], _CT,
        preferred_element_type=jnp.float32) + bias2_ref[...]
    o2_ref[...] = acc.astype(_BF)


def _attn_kernel(q_ref, k_ref, v_ref, o_ref):
    q = q_ref[...]
    k = k_ref[...]
    v = v_ref[...]
    scale = 1.0 / float(DH) ** 0.5
    for h in range(NH):
        lo, hi = h * DH, (h + 1) * DH
        qh = q[:, lo:hi]
        kh = k[:, lo:hi]
        vh = v[:, lo:hi]
        s = jax.lax.dot_general(
            qh, kh, _CT, preferred_element_type=jnp.float32) * scale
        p = jnp.exp(s - 12.0)
        inv = 1.0 / jnp.sum(p, axis=1, keepdims=True)
        o = jnp.dot(p.astype(_BF), vh, preferred_element_type=jnp.float32)
        o_ref[:, lo:hi] = (o * inv).astype(_BF)


def _gate_kernel(img_ref, txt_ref, q2_ref, iga_ref, igb_ref, igbias_ref,
                 tga_ref, tgb_ref, tgbias_ref, lns_ref, lnb_ref,
                 tp_ref, ip_ref, qffn_ref):
    img = img_ref[...]
    q2 = q2_ref[...]
    ictx = jnp.mean(img, axis=0, keepdims=True)
    tctx = jnp.mean(txt_ref[...], axis=0, keepdims=True)
    il = (
        jnp.dot(img.astype(_BF), iga_ref[...],
                preferred_element_type=jnp.float32)
        + jnp.dot(tctx.astype(_BF), igb_ref[...],
                  preferred_element_type=jnp.float32)
        + igbias_ref[...]
    )
    tl = (
        jnp.dot(q2.astype(_BF), tga_ref[...],
                preferred_element_type=jnp.float32)
        + jnp.dot(ictx.astype(_BF), tgb_ref[...],
                  preferred_element_type=jnp.float32)
        + tgbias_ref[...]
    )

    def _softmax(x):
        m = jnp.max(x, axis=1, keepdims=True)
        e = jnp.exp(x - m)
        return e / jnp.sum(e, axis=1, keepdims=True)

    ip_ref[...] = _softmax(il)
    tp_ref[...] = _softmax(tl)
    mu = jnp.mean(q2, axis=1, keepdims=True)
    var = jnp.mean((q2 - mu) ** 2, axis=1, keepdims=True)
    qffn = (q2 - mu) / jnp.sqrt(var + 1e-5) * lns_ref[...] + lnb_ref[...]
    qffn_ref[...] = qffn.astype(_BF)


def _moe_kernel(xq_ref, xi_ref, rq_ref, ri_ref, pq_ref, pi_ref,
                w1_ref, b1_ref, w2_ref, b2_ref,
                oq_ref, oi_ref, w1b_ref, w2b_ref):
    e = pl.program_id(0)
    w1b_ref[...] = w1_ref[0].astype(_BF)
    w2b_ref[...] = w2_ref[0].astype(_BF)
    iota = jax.lax.broadcasted_iota(jnp.int32, (E, 1), 0)
    onehot = (iota == e).astype(jnp.float32)

    for t in range(_NT):
        if t < _NTQ:
            lo, hi = t * _BTM, (t + 1) * _BTM
            x = xq_ref[lo:hi, :]
            probs = pq_ref[lo:hi, :]
            r_ref, o_ref = rq_ref, oq_ref
        else:
            lo, hi = (t - _NTQ) * _BTM, (t - _NTQ + 1) * _BTM
            x = xi_ref[lo:hi, :]
            probs = pi_ref[lo:hi, :]
            r_ref, o_ref = ri_ref, oi_ref
        h = jax.lax.dot_general(
            x, w1b_ref[...], _CT,
            preferred_element_type=jnp.float32).astype(_BF) + b1_ref[0]
        h = jax.nn.gelu(h)
        y = jax.lax.dot_general(
            h, w2b_ref[...], _CT,
            preferred_element_type=jnp.float32) + b2_ref[0]
        p = jnp.dot(probs, onehot, preferred_element_type=jnp.float32)
        contrib = p * y

        @pl.when(e == 0)
        def _():
            o_ref[lo:hi, :] = r_ref[lo:hi, :] + contrib

        @pl.when(e != 0)
        def _():
            o_ref[lo:hi, :] += contrib


def _lnmm(x, s, b, w, bias):
    n, k = x.shape
    m = w.shape[0]
    return pl.pallas_call(
        _lnmm_kernel,
        grid=(n // _BT,),
        in_specs=[
            pl.BlockSpec((_BT, k), lambda i: (i, 0)),
            pl.BlockSpec((1, k), lambda i: (0, 0)),
            pl.BlockSpec((1, k), lambda i: (0, 0)),
            pl.BlockSpec((m, k), lambda i: (0, 0)),
            pl.BlockSpec((1, m), lambda i: (0, 0)),
        ],
        out_specs=pl.BlockSpec((_BT, m), lambda i: (i, 0)),
        out_shape=jax.ShapeDtypeStruct((n, m), _BF),
        scratch_shapes=[pltpu.VMEM((m, k), _BF)],
    )(x, s[None], b[None], w, bias[None])


def _mm(x, w, bias):
    n, k = x.shape
    m = w.shape[0]
    return pl.pallas_call(
        _mm_kernel,
        grid=(n // _BT,),
        in_specs=[
            pl.BlockSpec((_BT, k), lambda i: (i, 0)),
            pl.BlockSpec((m, k), lambda i: (0, 0)),
            pl.BlockSpec((1, m), lambda i: (0, 0)),
        ],
        out_specs=pl.BlockSpec((_BT, m), lambda i: (i, 0)),
        out_shape=jax.ShapeDtypeStruct((n, m), _BF),
        scratch_shapes=[pltpu.VMEM((m, k), _BF)],
    )(x, w, bias[None])


def _mmres(x, w, bias, res):
    n, k = x.shape
    return pl.pallas_call(
        _mmres_kernel,
        grid=(n // _BT,),
        in_specs=[
            pl.BlockSpec((_BT, k), lambda i: (i, 0)),
            pl.BlockSpec((H, k), lambda i: (0, 0)),
            pl.BlockSpec((1, H), lambda i: (0, 0)),
            pl.BlockSpec((_BT, H), lambda i: (i, 0)),
        ],
        out_specs=pl.BlockSpec((_BT, H), lambda i: (i, 0)),
        out_shape=jax.ShapeDtypeStruct((n, H), jnp.float32),
        scratch_shapes=[pltpu.VMEM((H, k), _BF)],
    )(x, w, bias[None], res)


def _mmres_lnmm(x, w, bias, res, s, b, w2, bias2):
    n, k = x.shape
    return pl.pallas_call(
        _mmres_lnmm_kernel,
        grid=(n // _BT,),
        in_specs=[
            pl.BlockSpec((_BT, k), lambda i: (i, 0)),
            pl.BlockSpec((H, k), lambda i: (0, 0)),
            pl.BlockSpec((1, H), lambda i: (0, 0)),
            pl.BlockSpec((_BT, H), lambda i: (i, 0)),
            pl.BlockSpec((1, H), lambda i: (0, 0)),
            pl.BlockSpec((1, H), lambda i: (0, 0)),
            pl.BlockSpec((H, H), lambda i: (0, 0)),
            pl.BlockSpec((1, H), lambda i: (0, 0)),
        ],
        out_specs=[
            pl.BlockSpec((_BT, H), lambda i: (i, 0)),
            pl.BlockSpec((_BT, H), lambda i: (i, 0)),
        ],
        out_shape=[
            jax.ShapeDtypeStruct((n, H), jnp.float32),
            jax.ShapeDtypeStruct((n, H), _BF),
        ],
        scratch_shapes=[pltpu.VMEM((H, H), _BF), pltpu.VMEM((H, H), _BF)],
    )(x, w, bias[None], res, s[None], b[None], w2, bias2[None])


def _attn(q_arr, q_idx, kv_arr, k_idx, v_idx, sk):
    n = q_arr.shape[0]
    return pl.pallas_call(
        _attn_kernel,
        grid=(n // _BT,),
        in_specs=[
            pl.BlockSpec((_BT, H), lambda i: (i, q_idx)),
            pl.BlockSpec((sk, H), lambda i: (0, k_idx)),
            pl.BlockSpec((sk, H), lambda i: (0, v_idx)),
        ],
        out_specs=pl.BlockSpec((_BT, H), lambda i: (i, 0)),
        out_shape=jax.ShapeDtypeStruct((n, H), _BF),
    )(q_arr, kv_arr, kv_arr)


def _gate(img, txt, q2, ig_w, ig_b, tg_w, tg_b, ln_s, ln_b):
    si = img.shape[0]
    sq = q2.shape[0]
    return pl.pallas_call(
        _gate_kernel,
        out_shape=[
            jax.ShapeDtypeStruct((sq, E), jnp.float32),
            jax.ShapeDtypeStruct((si, E), jnp.float32),
            jax.ShapeDtypeStruct((sq, H), _BF),
        ],
    )(img, txt, q2,
      ig_w[:, :H].T.astype(_BF), ig_w[:, H:].T.astype(_BF), ig_b[None],
      tg_w[:, :H].T.astype(_BF), tg_w[:, H:].T.astype(_BF), tg_b[None],
      ln_s[None], ln_b[None])


def _moe(xq, xi, rq, ri, pq, pi, ew1, eb1, ew2, eb2):
    return pl.pallas_call(
        _moe_kernel,
        grid=(E,),
        in_specs=[
            pl.BlockSpec((SQ, H), lambda e: (0, 0)),
            pl.BlockSpec((SI, H), lambda e: (0, 0)),
            pl.BlockSpec((SQ, H), lambda e: (0, 0)),
            pl.BlockSpec((SI, H), lambda e: (0, 0)),
            pl.BlockSpec((SQ, E), lambda e: (0, 0)),
            pl.BlockSpec((SI, E), lambda e: (0, 0)),
            pl.BlockSpec((1, I, H), lambda e: (e, 0, 0)),
            pl.BlockSpec((1, 1, I), lambda e: (e, 0, 0)),
            pl.BlockSpec((1, H, I), lambda e: (e, 0, 0)),
            pl.BlockSpec((1, 1, H), lambda e: (e, 0, 0)),
        ],
        out_specs=[
            pl.BlockSpec((SQ, H), lambda e: (0, 0)),
            pl.BlockSpec((SI, H), lambda e: (0, 0)),
        ],
        out_shape=[
            jax.ShapeDtypeStruct((SQ, H), jnp.float32),
            jax.ShapeDtypeStruct((SI, H), jnp.float32),
        ],
        scratch_shapes=[
            pltpu.VMEM((I, H), _BF),
            pltpu.VMEM((H, I), _BF),
        ],
    )(xq, xi, rq, ri, pq, pi, ew1, eb1[:, None].astype(_BF), ew2,
      eb2[:, None])


def kernel(query_tokens, image_tokens, text_context, sa_w_in, sa_b_in,
           sa_w_out, sa_b_out, ca_w_in, ca_b_in, ca_w_out, ca_b_out,
           ln_q_s, ln_q_b, ln_c_s, ln_c_b, ln_f_s, ln_f_b,
           ig_w, ig_b, tg_w, tg_b, ew1, eb1, ew2, eb2):
    q = query_tokens[0]
    img = image_tokens[0]
    txt = text_context[0]

    qkv = _lnmm(q, ln_q_s, ln_q_b, sa_w_in, sa_b_in)
    attn = _attn(qkv, 0, qkv, 1, 2, q.shape[0])

    q1, cq = _mmres_lnmm(attn, sa_w_out, sa_b_out, q,
                         ln_c_s, ln_c_b, ca_w_in[:H], ca_b_in[:H])
    kv = _mm(img, ca_w_in[H:], ca_b_in[H:])
    cattn = _attn(cq, 0, kv, 0, 1, img.shape[0])
    q2 = _mmres(cattn, ca_w_out, ca_b_out, q1)

    tp, ip, qffn = _gate(img, txt, q2, ig_w, ig_b, tg_w, tg_b, ln_f_s, ln_f_b)

    q_out, img_out = _moe(qffn, img.astype(_BF), q2, img, tp, ip,
                          ew1, eb1, ew2, eb2)
    return (q_out[None], img_out[None])
